# baseline (speedup 1.0000x reference)
import functools

import jax
import jax.numpy as jnp
from jax import lax
from jax.experimental import pallas as pl
from jax.experimental.pallas import tpu as pltpu
from jax.experimental.pallas import tpu_sc as plsc

_VOCAB = 100000
_D = 768
_B = 4
_S = 8192
_BS = _B * _S

_NW = 32
_TPW = _BS // _NW
_NCH = 16
_C = _TPW // _NCH
_LG = _D // 16


@functools.partial(
    pl.kernel,
    mesh=plsc.VectorSubcoreMesh(core_axis_name="c", subcore_axis_name="s"),
    out_type=jax.ShapeDtypeStruct((_BS, _D), jnp.float32),
    scratch_types=[
        pltpu.VMEM((_NCH, _C), jnp.int32),
        pltpu.VMEM((_C, _D), jnp.float32),
        pltpu.VMEM((_C, _D), jnp.float32),
        pltpu.SemaphoreType.DMA,
    ],
)
def _emb_lookup(x_hbm, tok_hbm, pos_hbm, out_hbm, idx_v, tokbuf, posbuf, sem):
    cid = lax.axis_index("c")
    sid = lax.axis_index("s")
    wid = sid * 2 + cid
    base = wid * _TPW
    pos_base = lax.rem(base, _S)

    pltpu.sync_copy(x_hbm.at[wid], idx_v)

    def chunk_body(j, carry):
        pltpu.async_copy(tok_hbm.at[idx_v.at[j]], tokbuf, sem).wait()
        pltpu.sync_copy(pos_hbm.at[pl.ds(pos_base + j * _C, _C)], posbuf)

        def row_body(r, c2):
            for k in range(_LG):
                sl = pl.ds(k * 16, 16)
                tokbuf[r, sl] = tokbuf[r, sl] + posbuf[r, sl]
            return c2

        lax.fori_loop(0, _C, row_body, 0, unroll=False)
        pltpu.sync_copy(tokbuf, out_hbm.at[pl.ds(base + j * _C, _C)])
        return carry

    lax.fori_loop(0, _NCH, chunk_body, 0, unroll=False)


def kernel(x, token_table, pos_table):
    xf = x.astype(jnp.int32).reshape(_NW, _NCH, _C)
    out = _emb_lookup(xf, token_table, pos_table)
    return out.reshape(_B, _S, _D)


# speedup vs baseline: 1.6106x; 1.6106x over previous
import functools

import jax
import jax.numpy as jnp
from jax import lax
from jax.experimental import pallas as pl
from jax.experimental.pallas import tpu as pltpu
from jax.experimental.pallas import tpu_sc as plsc

_VOCAB = 100000
_D = 768
_B = 4
_S = 8192
_BS = _B * _S

_NW = 32
_SPW = _S // _NW
_NG = 8
_C = _SPW // _NG
_NCH = _NG * _B
_LG = _D // 16


@functools.partial(
    pl.kernel,
    mesh=plsc.VectorSubcoreMesh(core_axis_name="c", subcore_axis_name="s"),
    out_type=jax.ShapeDtypeStruct((_BS, _D), jnp.float32),
    scratch_types=[
        pltpu.VMEM((_NCH, _C), jnp.int32),
        pltpu.VMEM((_C, _D), jnp.float32),
        pltpu.VMEM((_C, _D), jnp.float32),
        pltpu.VMEM((_C, _D), jnp.float32),
        pltpu.SemaphoreType.DMA,
        pltpu.SemaphoreType.DMA,
    ],
)
def _emb_lookup(x_hbm, tok_hbm, pos_hbm, out_hbm, idx_v, tok0, tok1, posb,
                gsem, osem):
    cid = lax.axis_index("c")
    sid = lax.axis_index("s")
    wid = sid * 2 + cid
    s_base = wid * _SPW
    toks = (tok0, tok1)

    pltpu.sync_copy(x_hbm.at[wid], idx_v)

    def gather(t, buf):
        return pltpu.async_copy(tok_hbm.at[idx_v.at[t]], buf, gsem)

    def drain_gather(buf):
        pltpu.make_async_copy(tok_hbm.at[idx_v.at[0]], buf, gsem).wait()

    def out_write(g, b, buf):
        row = b * _S + s_base + g * _C
        return pltpu.async_copy(buf, out_hbm.at[pl.ds(row, _C)], osem)

    def drain_write(buf):
        pltpu.make_async_copy(buf, out_hbm.at[pl.ds(0, _C)], osem).wait()

    gather(0, tok0)

    def group_body(g, carry):
        pltpu.sync_copy(pos_hbm.at[pl.ds(s_base + g * _C, _C)], posb)
        for b in range(_B):
            t = g * _B + b
            cur = toks[b % 2]
            nxt = toks[1 - b % 2]
            @pl.when(t >= 1)
            def _():
                drain_write(nxt)

            @pl.when(t + 1 < _NCH)
            def _():
                gather(t + 1, nxt)

            drain_gather(cur)

            def row_body(r, c2):
                for k in range(_LG):
                    sl = pl.ds(k * 16, 16)
                    plsc.addupdate(cur.at[r, sl], posb[r, sl])
                return c2

            lax.fori_loop(0, _C, row_body, 0, unroll=False)
            out_write(g, b, cur)
        return carry

    lax.fori_loop(0, _NG, group_body, 0, unroll=False)
    drain_write(toks[(_NCH - 1) % 2])


def kernel(x, token_table, pos_table):
    xr = x.astype(jnp.int32).reshape(_B, _NW, _NG, _C)
    xr = xr.transpose(1, 2, 0, 3).reshape(_NW, _NCH, _C)
    out = _emb_lookup(xr, token_table, pos_table)
    return out.reshape(_B, _S, _D)


# speedup vs baseline: 1.8440x; 1.1449x over previous
import functools

import jax
import jax.numpy as jnp
from jax import lax
from jax.experimental import pallas as pl
from jax.experimental.pallas import tpu as pltpu
from jax.experimental.pallas import tpu_sc as plsc

_VOCAB = 100000
_D = 768
_B = 4
_S = 8192
_BS = _B * _S

_NW = 32
_SPW = _S // _NW
_NG = 16
_C = _SPW // _NG
_NCH = _NG * _B
_LG = _D // 16


@functools.partial(
    pl.kernel,
    mesh=plsc.VectorSubcoreMesh(core_axis_name="c", subcore_axis_name="s"),
    out_type=jax.ShapeDtypeStruct((_BS, _D), jnp.float32),
    scratch_types=[
        pltpu.VMEM((_NCH, _C), jnp.int32),
        pltpu.VMEM((2, _B, _C, _D), jnp.float32),
        pltpu.VMEM((2, _C, _D), jnp.float32),
        pltpu.SemaphoreType.DMA,
        pltpu.SemaphoreType.DMA,
        pltpu.SemaphoreType.DMA,
    ],
)
def _emb_lookup(x_hbm, tok_hbm, pos_hbm, out_hbm, idx_v, tokb, posb,
                psem, gsem, osem):
    cid = lax.axis_index("c")
    sid = lax.axis_index("s")
    wid = sid * 2 + cid
    s_base = wid * _SPW

    pltpu.sync_copy(x_hbm.at[wid], idx_v)

    def issue_group(g, par):
        pltpu.async_copy(pos_hbm.at[pl.ds(s_base + g * _C, _C)],
                         posb.at[par], psem)
        for b in range(_B):
            pltpu.async_copy(tok_hbm.at[idx_v.at[g * _B + b]],
                             tokb.at[par, b], gsem)

    def drain(sem, shaped):
        pltpu.make_async_copy(pos_hbm.at[pl.ds(0, _C)], shaped, sem).wait()

    issue_group(0, 0)

    def consume(g, par):
        @pl.when(g + 1 < _NG)
        def _():
            @pl.when(g >= 1)
            def _():
                for b in range(_B):
                    drain(osem, tokb.at[1 - par, b])
            issue_group(g + 1, 1 - par)

        drain(psem, posb.at[par])
        for b in range(_B):
            drain(gsem, tokb.at[par, b])

        def row_body(r, c2):
            for k in range(_LG):
                sl = pl.ds(k * 16, 16)
                pv = posb[par, r, sl]
                for b in range(_B):
                    plsc.addupdate(tokb.at[par, b, r, sl], pv)
            return c2

        lax.fori_loop(0, _C, row_body, 0, unroll=False)
        for b in range(_B):
            row = b * _S + s_base + g * _C
            pltpu.async_copy(tokb.at[par, b], out_hbm.at[pl.ds(row, _C)],
                             osem)

    def pair_body(gg, carry):
        consume(2 * gg, 0)
        consume(2 * gg + 1, 1)
        return carry

    lax.fori_loop(0, _NG // 2, pair_body, 0, unroll=False)
    for b in range(_B):
        drain(osem, tokb.at[1, b])


def kernel(x, token_table, pos_table):
    xr = x.astype(jnp.int32).reshape(_B, _NW, _NG, _C)
    xr = xr.transpose(1, 2, 0, 3).reshape(_NW, _NCH, _C)
    out = _emb_lookup(xr, token_table, pos_table)
    return out.reshape(_B, _S, _D)
